# Initial kernel scaffold; baseline (speedup 1.0000x reference)
import jax
import jax.numpy as jnp
from jax.experimental import pallas as pl


def kernel(x, Wqkv, bqkv, Wout, bout, gamma, beta):
    raise NotImplementedError("write your pallas kernel here")


# speedup vs baseline: 48.3507x; 48.3507x over previous
import functools

import jax
import jax.numpy as jnp
import numpy as np
from jax.experimental import pallas as pl
from jax.experimental.pallas import tpu as pltpu

D = 768
S = 2048
B = 4
K_KEEP = 614
_SCALE = 1.0 / np.sqrt(np.float32(D))

_RQ = 512
_RA = 256


def _qkv_body(x_ref, w_ref, b_ref, g_ref, be_ref, xn_ref, q_ref, k_ref, v_ref):
    x = x_ref[...]
    mu = jnp.mean(x, axis=-1, keepdims=True)
    var = jnp.mean((x - mu) * (x - mu), axis=-1, keepdims=True)
    xn = (x - mu) * jax.lax.rsqrt(var + 1e-5) * g_ref[...] + be_ref[...]
    xn_ref[...] = xn
    qkv = jax.lax.dot_general(xn, w_ref[...], (((1,), (1,)), ((), ())),
                              preferred_element_type=jnp.float32) + b_ref[...]
    q_ref[...] = qkv[:, :D]
    k_ref[...] = qkv[:, D:2 * D]
    v_ref[...] = qkv[:, 2 * D:]


def _attn_body(q_ref, k_ref, v_ref, xn_ref, wo_ref, bo_ref, o_ref):
    q = q_ref[0]
    k = k_ref[0]
    s = jax.lax.dot_general(q, k, (((1,), (1,)), ((), ())),
                            preferred_element_type=jnp.float32) * _SCALE
    y = jax.lax.bitcast_convert_type(s, jnp.int32)
    y = jnp.where(y < 0, y ^ jnp.int32(0x7FFFFFFF), y)

    lo = jnp.min(y, axis=-1, keepdims=True)
    hi = jnp.max(y, axis=-1, keepdims=True)

    def body(_, carry):
        lo, hi = carry
        mid = (lo >> 1) + (hi >> 1) + ((lo | hi) & 1)
        cnt = jnp.sum((y >= mid).astype(jnp.int32), axis=-1, keepdims=True)
        ge = cnt >= K_KEEP
        return jnp.where(ge, mid, lo), jnp.where(ge, hi, mid - 1)

    lo, hi = jax.lax.fori_loop(0, 32, body, (lo, hi))

    mask = y >= lo
    m = jnp.max(s, axis=-1, keepdims=True)
    p = jnp.where(mask, jnp.exp(s - m), 0.0)
    z = jnp.sum(p, axis=-1, keepdims=True)
    w = p / z
    attn = jax.lax.dot_general(w, v_ref[0], (((1,), (0,)), ((), ())),
                               preferred_element_type=jnp.float32)
    out = jax.lax.dot_general(attn, wo_ref[...], (((1,), (1,)), ((), ())),
                              preferred_element_type=jnp.float32) + bo_ref[...]
    o_ref[0] = out + xn_ref[0]


@jax.jit
def kernel(x, Wqkv, bqkv, Wout, bout, gamma, beta):
    xf = x.reshape(B * S, D)
    xn, q, k, v = pl.pallas_call(
        _qkv_body,
        grid=(B * S // _RQ,),
        in_specs=[
            pl.BlockSpec((_RQ, D), lambda i: (i, 0)),
            pl.BlockSpec((3 * D, D), lambda i: (0, 0)),
            pl.BlockSpec((1, 3 * D), lambda i: (0, 0)),
            pl.BlockSpec((1, D), lambda i: (0, 0)),
            pl.BlockSpec((1, D), lambda i: (0, 0)),
        ],
        out_specs=[
            pl.BlockSpec((_RQ, D), lambda i: (i, 0)),
            pl.BlockSpec((_RQ, D), lambda i: (i, 0)),
            pl.BlockSpec((_RQ, D), lambda i: (i, 0)),
            pl.BlockSpec((_RQ, D), lambda i: (i, 0)),
        ],
        out_shape=[jax.ShapeDtypeStruct((B * S, D), jnp.float32)] * 4,
    )(xf, Wqkv, bqkv.reshape(1, 3 * D), gamma.reshape(1, D),
      beta.reshape(1, D))

    q = q.reshape(B, S, D)
    k = k.reshape(B, S, D)
    v = v.reshape(B, S, D)
    xn = xn.reshape(B, S, D)

    out = pl.pallas_call(
        _attn_body,
        grid=(B, S // _RA),
        in_specs=[
            pl.BlockSpec((1, _RA, D), lambda b, i: (b, i, 0)),
            pl.BlockSpec((1, S, D), lambda b, i: (b, 0, 0)),
            pl.BlockSpec((1, S, D), lambda b, i: (b, 0, 0)),
            pl.BlockSpec((1, _RA, D), lambda b, i: (b, i, 0)),
            pl.BlockSpec((D, D), lambda b, i: (0, 0)),
            pl.BlockSpec((1, D), lambda b, i: (0, 0)),
        ],
        out_specs=pl.BlockSpec((1, _RA, D), lambda b, i: (b, i, 0)),
        out_shape=jax.ShapeDtypeStruct((B, S, D), jnp.float32),
    )(q, k, v, xn, Wout, bout.reshape(1, D))

    return out


# speedup vs baseline: 49.2268x; 1.0181x over previous
import functools

import jax
import jax.numpy as jnp
import numpy as np
from jax.experimental import pallas as pl
from jax.experimental.pallas import tpu as pltpu

D = 768
S = 2048
B = 4
K_KEEP = 614
_SCALE = 1.0 / np.sqrt(np.float32(D))

_RQ = 512
_RA = 256


def _qkv_body(x_ref, w_ref, b_ref, g_ref, be_ref, xn_ref, q_ref, k_ref, v_ref):
    x = x_ref[...]
    mu = jnp.mean(x, axis=-1, keepdims=True)
    var = jnp.mean((x - mu) * (x - mu), axis=-1, keepdims=True)
    xn = (x - mu) * jax.lax.rsqrt(var + 1e-5) * g_ref[...] + be_ref[...]
    xn_ref[...] = xn
    qkv = jax.lax.dot_general(xn.astype(jnp.bfloat16), w_ref[...],
                              (((1,), (1,)), ((), ())),
                              preferred_element_type=jnp.float32) + b_ref[...]
    q_ref[...] = qkv[:, :D].astype(jnp.bfloat16)
    k_ref[...] = qkv[:, D:2 * D].astype(jnp.bfloat16)
    v_ref[...] = qkv[:, 2 * D:].astype(jnp.bfloat16)


def _attn_body(q_ref, k_ref, v_ref, xn_ref, wo_ref, bo_ref, o_ref):
    q = q_ref[0]
    k = k_ref[0]
    s = jax.lax.dot_general(q, k, (((1,), (1,)), ((), ())),
                            preferred_element_type=jnp.float32) * _SCALE
    y = jax.lax.bitcast_convert_type(s, jnp.int32)
    y = jnp.where(y < 0, y ^ jnp.int32(0x7FFFFFFF), y)

    lo = jnp.min(y, axis=-1, keepdims=True)
    hi = jnp.max(y, axis=-1, keepdims=True)

    def body(_, carry):
        lo, hi = carry
        mid = (lo >> 1) + (hi >> 1) + ((lo | hi) & 1)
        cnt = jnp.sum((y >= mid).astype(jnp.int32), axis=-1, keepdims=True)
        ge = cnt >= K_KEEP
        return jnp.where(ge, mid, lo), jnp.where(ge, hi, mid - 1)

    lo, hi = jax.lax.fori_loop(0, 32, body, (lo, hi))

    mask = y >= lo
    m = jnp.max(s, axis=-1, keepdims=True)
    p = jnp.where(mask, jnp.exp(s - m), 0.0)
    z = jnp.sum(p, axis=-1, keepdims=True)
    w = (p / z).astype(jnp.bfloat16)
    attn = jax.lax.dot_general(w, v_ref[0], (((1,), (0,)), ((), ())),
                               preferred_element_type=jnp.float32)
    out = jax.lax.dot_general(attn.astype(jnp.bfloat16), wo_ref[...],
                              (((1,), (1,)), ((), ())),
                              preferred_element_type=jnp.float32) + bo_ref[...]
    o_ref[0] = out + xn_ref[0]


@jax.jit
def kernel(x, Wqkv, bqkv, Wout, bout, gamma, beta):
    xf = x.reshape(B * S, D)
    xn, q, k, v = pl.pallas_call(
        _qkv_body,
        grid=(B * S // _RQ,),
        in_specs=[
            pl.BlockSpec((_RQ, D), lambda i: (i, 0)),
            pl.BlockSpec((3 * D, D), lambda i: (0, 0)),
            pl.BlockSpec((1, 3 * D), lambda i: (0, 0)),
            pl.BlockSpec((1, D), lambda i: (0, 0)),
            pl.BlockSpec((1, D), lambda i: (0, 0)),
        ],
        out_specs=[
            pl.BlockSpec((_RQ, D), lambda i: (i, 0)),
            pl.BlockSpec((_RQ, D), lambda i: (i, 0)),
            pl.BlockSpec((_RQ, D), lambda i: (i, 0)),
            pl.BlockSpec((_RQ, D), lambda i: (i, 0)),
        ],
        out_shape=[jax.ShapeDtypeStruct((B * S, D), jnp.float32)] +
                  [jax.ShapeDtypeStruct((B * S, D), jnp.bfloat16)] * 3,
    )(xf, Wqkv.astype(jnp.bfloat16), bqkv.reshape(1, 3 * D),
      gamma.reshape(1, D), beta.reshape(1, D))

    q = q.reshape(B, S, D)
    k = k.reshape(B, S, D)
    v = v.reshape(B, S, D)
    xn = xn.reshape(B, S, D)

    out = pl.pallas_call(
        _attn_body,
        grid=(B, S // _RA),
        in_specs=[
            pl.BlockSpec((1, _RA, D), lambda b, i: (b, i, 0)),
            pl.BlockSpec((1, S, D), lambda b, i: (b, 0, 0)),
            pl.BlockSpec((1, S, D), lambda b, i: (b, 0, 0)),
            pl.BlockSpec((1, _RA, D), lambda b, i: (b, i, 0)),
            pl.BlockSpec((D, D), lambda b, i: (0, 0)),
            pl.BlockSpec((1, D), lambda b, i: (0, 0)),
        ],
        out_specs=pl.BlockSpec((1, _RA, D), lambda b, i: (b, i, 0)),
        out_shape=jax.ShapeDtypeStruct((B, S, D), jnp.float32),
    )(q, k, v, xn, Wout.astype(jnp.bfloat16), bout.reshape(1, D))

    return out
